# Initial kernel scaffold; baseline (speedup 1.0000x reference)
import jax
import jax.numpy as jnp
from jax.experimental import pallas as pl


def kernel(x_flat, h_all, V):
    raise NotImplementedError("write your pallas kernel here")


# speedup vs baseline: 3.9255x; 3.9255x over previous
import functools
import math

import jax
import jax.numpy as jnp
from jax.experimental import pallas as pl

_K = 2
_EPS = 1e-08
_TN = 256
_HI = jax.lax.Precision.HIGHEST


def _dot(a, b):
    return jax.lax.dot(a, b, precision=_HI, preferred_element_type=jnp.float32)


def _body(n_grid, n_tokens, m_experts, b_code, x_ref, h_ref, vt_ref,
          hs_ref, idx_ref, scal_ref):
    g = pl.program_id(0)
    mb = m_experts * b_code
    h = h_ref[...]
    hh = h * h

    jrow = jax.lax.broadcasted_iota(jnp.int32, (mb, m_experts), 0)
    mcol = jax.lax.broadcasted_iota(jnp.int32, (mb, m_experts), 1)
    s_mat = (jrow // b_code == mcol).astype(jnp.float32)
    energy = _dot(hh, s_mat)

    iota_m = jax.lax.broadcasted_iota(jnp.int32, (h.shape[0], m_experts), 1)
    e0 = jnp.max(energy, axis=1, keepdims=True)
    m0 = jnp.min(jnp.where(energy == e0, iota_m, m_experts), axis=1,
                 keepdims=True)
    masked = jnp.where(iota_m == m0, -jnp.inf, energy)
    e1 = jnp.max(masked, axis=1, keepdims=True)
    m1 = jnp.min(jnp.where(masked == e1, iota_m, m_experts), axis=1,
                 keepdims=True)

    oh0 = (iota_m == m0).astype(jnp.float32)
    oh1 = (iota_m == m1).astype(jnp.float32)

    mrow = jax.lax.broadcasted_iota(jnp.int32, (m_experts, mb), 0)
    jcol = jax.lax.broadcasted_iota(jnp.int32, (m_experts, mb), 1)
    st_mat = (jcol // b_code == mrow).astype(jnp.float32)

    w_exp = _dot(oh0 + oh1, st_mat)
    h_masked = h * w_exp
    x_hat = _dot(h_masked, vt_ref[...])
    x = x_ref[...]
    resid = x - x_hat

    j2 = jax.lax.broadcasted_iota(jnp.int32, (mb, _K * b_code), 0)
    c2 = jax.lax.broadcasted_iota(jnp.int32, (mb, _K * b_code), 1)
    ra = (j2 % b_code == c2).astype(jnp.float32)
    rb = (j2 % b_code == c2 - b_code).astype(jnp.float32)
    hs = _dot(h * _dot(oh0, st_mat), ra) + _dot(h * _dot(oh1, st_mat), rb)
    hs_ref[...] = hs

    iota_k = jax.lax.broadcasted_iota(jnp.int32, (h.shape[0], _K), 1)
    idx_ref[...] = jnp.where(iota_k == 0, m0, m1)

    cap_s = jnp.sum(e0 + e1)
    rec_s = jnp.sum(x_hat * x_hat)
    unc_s = jnp.sum(resid * resid)
    esum = jnp.sum(energy, axis=0, keepdims=True)
    prow = jax.lax.broadcasted_iota(jnp.int32, (m_experts, 128), 0)
    pcol = jax.lax.broadcasted_iota(jnp.int32, (m_experts, 128), 1)
    p_mat = (pcol == prow + 8).astype(jnp.float32)
    il = jax.lax.broadcasted_iota(jnp.int32, (1, 128), 1)
    stepvec = ((il == 0).astype(jnp.float32) * cap_s
               + (il == 1).astype(jnp.float32) * rec_s
               + (il == 2).astype(jnp.float32) * unc_s
               + _dot(esum, p_mat))

    @pl.when(g == 0)
    def _():
        scal_ref[...] = stepvec

    @pl.when(g > 0)
    def _():
        scal_ref[...] = scal_ref[...] + stepvec

    @pl.when(g == n_grid - 1)
    def _():
        acc = scal_ref[...]
        n_f = float(n_tokens)
        emask = ((il >= 8) & (il < 8 + m_experts)).astype(jnp.float32)
        avg = acc * emask / n_f
        denom = jnp.maximum(jnp.sum(avg), _EPS)
        probs = jnp.maximum(avg / denom, _EPS)
        ent = -jnp.sum(emask * probs * jnp.log(probs)) / math.log(m_experts)
        cap = jnp.sum(acc * (il == 0).astype(jnp.float32)) / n_f
        rec = jnp.sum(acc * (il == 1).astype(jnp.float32)) / n_f
        unc = jnp.sum(acc * (il == 2).astype(jnp.float32)) / n_f
        aux = unc + 0.5 * (1.0 - ent)
        scal_ref[...] = ((il == 0).astype(jnp.float32) * cap
                         + (il == 1).astype(jnp.float32) * rec
                         + (il == 2).astype(jnp.float32) * unc
                         + (il == 3).astype(jnp.float32) * ent
                         + (il == 4).astype(jnp.float32) * aux)


@functools.partial(jax.jit, static_argnames=("interpret",))
def kernel(x_flat, h_all, V, interpret=False):
    n, d = x_flat.shape
    _, m, b = h_all.shape
    mb = m * b
    h2 = h_all.reshape(n, mb)
    vt = V.reshape(d, mb).T
    tn = min(_TN, n)
    n_grid = n // tn

    body = functools.partial(_body, n_grid, n, m, b)
    hs, idx, scal = pl.pallas_call(
        body,
        grid=(n_grid,),
        in_specs=[
            pl.BlockSpec((tn, d), lambda g: (g, 0)),
            pl.BlockSpec((tn, mb), lambda g: (g, 0)),
            pl.BlockSpec((mb, d), lambda g: (0, 0)),
        ],
        out_specs=[
            pl.BlockSpec((tn, _K * b), lambda g: (g, 0)),
            pl.BlockSpec((tn, _K), lambda g: (g, 0)),
            pl.BlockSpec((1, 128), lambda g: (0, 0)),
        ],
        out_shape=[
            jax.ShapeDtypeStruct((n, _K * b), jnp.float32),
            jax.ShapeDtypeStruct((n, _K), jnp.int32),
            jax.ShapeDtypeStruct((1, 128), jnp.float32),
        ],
        interpret=interpret,
    )(x_flat, h2, vt)

    return (hs.reshape(n, _K, b), idx, scal[0, 0], scal[0, 1], scal[0, 2],
            scal[0, 3], scal[0, 4])


# speedup vs baseline: 7.3804x; 1.8801x over previous
import functools
import math

import numpy as np
import jax
import jax.numpy as jnp
from jax.experimental import pallas as pl

_K = 2
_EPS = 1e-08
_TN = 512
_HI = jax.lax.Precision.HIGHEST


def _dot(a, b, prec=None):
    return jax.lax.dot(a, b, precision=prec, preferred_element_type=jnp.float32)


def _body(n_grid, n_tokens, m_experts, b_code, x_ref, h_ref, vt_ref,
          s_ref, st_ref, ra_ref, rb_ref, p_ref, hs_ref, idx_ref, scal_ref):
    g = pl.program_id(0)
    h = h_ref[...]
    energy = _dot(h * h, s_ref[...], _HI)

    iota_m = jax.lax.broadcasted_iota(jnp.int32, (h.shape[0], m_experts), 1)
    e0 = jnp.max(energy, axis=1, keepdims=True)
    m0 = jnp.min(jnp.where(energy == e0, iota_m, m_experts), axis=1,
                 keepdims=True)
    masked = jnp.where(iota_m == m0, -jnp.inf, energy)
    e1 = jnp.max(masked, axis=1, keepdims=True)
    m1 = jnp.min(jnp.where(masked == e1, iota_m, m_experts), axis=1,
                 keepdims=True)

    oh0 = (iota_m == m0).astype(jnp.float32)
    oh1 = (iota_m == m1).astype(jnp.float32)

    st = st_ref[...]
    a0 = h * _dot(oh0, st)
    a1 = h * _dot(oh1, st)
    h_masked = a0 + a1
    x_hat = _dot(h_masked, vt_ref[...])
    x = x_ref[...]
    resid = x - x_hat

    hs_ref[...] = _dot(a0, ra_ref[...]) + _dot(a1, rb_ref[...])

    iota_k = jax.lax.broadcasted_iota(jnp.int32, (h.shape[0], _K), 1)
    idx_ref[...] = jnp.where(iota_k == 0, m0, m1)

    cap_s = jnp.sum(e0 + e1)
    rec_s = jnp.sum(x_hat * x_hat)
    unc_s = jnp.sum(resid * resid)
    esum = jnp.sum(energy, axis=0, keepdims=True)
    il = jax.lax.broadcasted_iota(jnp.int32, (1, 128), 1)
    stepvec = ((il == 0).astype(jnp.float32) * cap_s
               + (il == 1).astype(jnp.float32) * rec_s
               + (il == 2).astype(jnp.float32) * unc_s
               + _dot(esum, p_ref[...]))

    @pl.when(g == 0)
    def _():
        scal_ref[...] = stepvec

    @pl.when(g > 0)
    def _():
        scal_ref[...] = scal_ref[...] + stepvec

    @pl.when(g == n_grid - 1)
    def _():
        acc = scal_ref[...]
        n_f = float(n_tokens)
        emask = ((il >= 8) & (il < 8 + m_experts)).astype(jnp.float32)
        avg = acc * emask / n_f
        denom = jnp.maximum(jnp.sum(avg), _EPS)
        probs = jnp.maximum(avg / denom, _EPS)
        ent = -jnp.sum(emask * probs * jnp.log(probs)) / math.log(m_experts)
        cap = jnp.sum(acc * (il == 0).astype(jnp.float32)) / n_f
        rec = jnp.sum(acc * (il == 1).astype(jnp.float32)) / n_f
        unc = jnp.sum(acc * (il == 2).astype(jnp.float32)) / n_f
        aux = unc + 0.5 * (1.0 - ent)
        scal_ref[...] = ((il == 0).astype(jnp.float32) * cap
                         + (il == 1).astype(jnp.float32) * rec
                         + (il == 2).astype(jnp.float32) * unc
                         + (il == 3).astype(jnp.float32) * ent
                         + (il == 4).astype(jnp.float32) * aux)


@functools.partial(jax.jit, static_argnames=("interpret",))
def kernel(x_flat, h_all, V, interpret=False):
    n, d = x_flat.shape
    _, m, b = h_all.shape
    mb = m * b
    h2 = h_all.reshape(n, mb)
    vt = V.reshape(d, mb).T
    tn = min(_TN, n)
    n_grid = n // tn

    j = np.arange(mb)
    s_np = (j[:, None] // b == np.arange(m)[None, :]).astype(np.float32)
    st_np = s_np.T.copy()
    c = np.arange(_K * b)
    ra_np = (j[:, None] % b == c[None, :]).astype(np.float32)
    rb_np = (j[:, None] % b == c[None, :] - b).astype(np.float32)
    p_np = (np.arange(m)[:, None] + 8 == np.arange(128)[None, :]).astype(
        np.float32)

    body = functools.partial(_body, n_grid, n, m, b)
    hs, idx, scal = pl.pallas_call(
        body,
        grid=(n_grid,),
        in_specs=[
            pl.BlockSpec((tn, d), lambda g: (g, 0)),
            pl.BlockSpec((tn, mb), lambda g: (g, 0)),
            pl.BlockSpec((mb, d), lambda g: (0, 0)),
            pl.BlockSpec((mb, m), lambda g: (0, 0)),
            pl.BlockSpec((m, mb), lambda g: (0, 0)),
            pl.BlockSpec((mb, _K * b), lambda g: (0, 0)),
            pl.BlockSpec((mb, _K * b), lambda g: (0, 0)),
            pl.BlockSpec((m, 128), lambda g: (0, 0)),
        ],
        out_specs=[
            pl.BlockSpec((tn, _K * b), lambda g: (g, 0)),
            pl.BlockSpec((tn, _K), lambda g: (g, 0)),
            pl.BlockSpec((1, 128), lambda g: (0, 0)),
        ],
        out_shape=[
            jax.ShapeDtypeStruct((n, _K * b), jnp.float32),
            jax.ShapeDtypeStruct((n, _K), jnp.int32),
            jax.ShapeDtypeStruct((1, 128), jnp.float32),
        ],
        interpret=interpret,
    )(x_flat, h2, vt, s_np, st_np, ra_np, rb_np, p_np)

    return (hs.reshape(n, _K, b), idx, scal[0, 0], scal[0, 1], scal[0, 2],
            scal[0, 3], scal[0, 4])


# speedup vs baseline: 7.9512x; 1.0773x over previous
import functools
import math

import numpy as np
import jax
import jax.numpy as jnp
from jax.experimental import pallas as pl

_K = 2
_EPS = 1e-08
_TN = 512
_HI = jax.lax.Precision.HIGHEST


def _dot(a, b, prec=None):
    return jax.lax.dot(a, b, precision=prec, preferred_element_type=jnp.float32)


def _body(n_grid, n_tokens, m_experts, b_code, x_ref, h_ref, vt_ref,
          s_ref, p_ref, hs_ref, idx_ref, scal_ref):
    g = pl.program_id(0)
    h = h_ref[...]
    energy = _dot(h * h, s_ref[...], _HI)

    iota_m = jax.lax.broadcasted_iota(jnp.int32, (h.shape[0], m_experts), 1)
    eb = jax.lax.bitcast_convert_type(energy, jnp.int32)
    key = (eb & ~15) | (15 - iota_m)
    k0 = jnp.max(key, axis=1, keepdims=True)
    m0 = 15 - (k0 & 15)
    e0 = jax.lax.bitcast_convert_type(k0 & ~15, jnp.float32)
    key2 = jnp.where(iota_m == m0, -1, key)
    k1 = jnp.max(key2, axis=1, keepdims=True)
    m1 = 15 - (k1 & 15)
    e1 = jax.lax.bitcast_convert_type(k1 & ~15, jnp.float32)

    mb = m_experts * b_code
    jexp = jax.lax.broadcasted_iota(jnp.int32, (h.shape[0], mb), 1) // b_code
    a0 = jnp.where(jexp == m0, h, 0.0)
    a1 = jnp.where(jexp == m1, h, 0.0)
    h_masked = (a0 + a1).astype(jnp.bfloat16)
    x_hat = _dot(h_masked, vt_ref[...])
    x = x_ref[...]
    resid = x - x_hat

    def _fold(t):
        w = t.shape[1]
        while w > b_code:
            w //= 2
            t = t[:, :w] + t[:, w:]
        return t

    hs_ref[...] = jnp.concatenate([_fold(a0), _fold(a1)], axis=1)

    iota_k = jax.lax.broadcasted_iota(jnp.int32, (h.shape[0], _K), 1)
    idx_ref[...] = jnp.where(iota_k == 0, m0, m1)

    cap_s = jnp.sum(e0 + e1)
    rec_s = jnp.sum(x_hat * x_hat)
    unc_s = jnp.sum(resid * resid)
    esum = jnp.sum(energy, axis=0, keepdims=True)
    il = jax.lax.broadcasted_iota(jnp.int32, (1, 128), 1)
    stepvec = ((il == 0).astype(jnp.float32) * cap_s
               + (il == 1).astype(jnp.float32) * rec_s
               + (il == 2).astype(jnp.float32) * unc_s
               + _dot(esum, p_ref[...]))

    @pl.when(g == 0)
    def _():
        scal_ref[...] = stepvec

    @pl.when(g > 0)
    def _():
        scal_ref[...] = scal_ref[...] + stepvec

    @pl.when(g == n_grid - 1)
    def _():
        acc = scal_ref[...]
        n_f = float(n_tokens)
        emask = ((il >= 8) & (il < 8 + m_experts)).astype(jnp.float32)
        avg = acc * emask / n_f
        denom = jnp.maximum(jnp.sum(avg), _EPS)
        probs = jnp.maximum(avg / denom, _EPS)
        ent = -jnp.sum(emask * probs * jnp.log(probs)) / math.log(m_experts)
        cap = jnp.sum(acc * (il == 0).astype(jnp.float32)) / n_f
        rec = jnp.sum(acc * (il == 1).astype(jnp.float32)) / n_f
        unc = jnp.sum(acc * (il == 2).astype(jnp.float32)) / n_f
        aux = unc + 0.5 * (1.0 - ent)
        scal_ref[...] = ((il == 0).astype(jnp.float32) * cap
                         + (il == 1).astype(jnp.float32) * rec
                         + (il == 2).astype(jnp.float32) * unc
                         + (il == 3).astype(jnp.float32) * ent
                         + (il == 4).astype(jnp.float32) * aux)


@functools.partial(jax.jit, static_argnames=("interpret",))
def kernel(x_flat, h_all, V, interpret=False):
    n, d = x_flat.shape
    _, m, b = h_all.shape
    mb = m * b
    h2 = h_all.reshape(n, mb)
    vt = V.reshape(d, mb).T.astype(jnp.bfloat16)
    tn = min(_TN, n)
    n_grid = n // tn

    j = np.arange(mb)
    s_np = (j[:, None] // b == np.arange(m)[None, :]).astype(np.float32)
    p_np = (np.arange(m)[:, None] + 8 == np.arange(128)[None, :]).astype(
        np.float32)

    body = functools.partial(_body, n_grid, n, m, b)
    hs, idx, scal = pl.pallas_call(
        body,
        grid=(n_grid,),
        in_specs=[
            pl.BlockSpec((tn, d), lambda g: (g, 0)),
            pl.BlockSpec((tn, mb), lambda g: (g, 0)),
            pl.BlockSpec((mb, d), lambda g: (0, 0)),
            pl.BlockSpec((mb, m), lambda g: (0, 0)),
            pl.BlockSpec((m, 128), lambda g: (0, 0)),
        ],
        out_specs=[
            pl.BlockSpec((tn, _K * b), lambda g: (g, 0)),
            pl.BlockSpec((tn, _K), lambda g: (g, 0)),
            pl.BlockSpec((1, 128), lambda g: (0, 0)),
        ],
        out_shape=[
            jax.ShapeDtypeStruct((n, _K * b), jnp.float32),
            jax.ShapeDtypeStruct((n, _K), jnp.int32),
            jax.ShapeDtypeStruct((1, 128), jnp.float32),
        ],
        interpret=interpret,
    )(x_flat, h2, vt, s_np, p_np)

    return (hs.reshape(n, _K, b), idx, scal[0, 0], scal[0, 1], scal[0, 2],
            scal[0, 3], scal[0, 4])
